# baseline (speedup 1.0000x reference)
import jax
import jax.numpy as jnp
from jax import lax
from jax.experimental import pallas as pl
from jax.experimental.pallas import tpu as pltpu
from jax.experimental.pallas import tpu_sc as plsc

_NC, _NS = 2, 16
_NW = _NC * _NS
_TB = 256


def _tc_copy_body(msg_ref, ent_ref, sums_ref, out_ref, sent_ref, snn_ref,
                  ment_ref, mnn_ref):
    out_ref[...] = msg_ref[...]
    e = ent_ref[...]
    sent_ref[...] = e
    snn_ref[...] = e
    s = sums_ref[...]
    ment_ref[...] = s
    mnn_ref[...] = s


def _sc_sums_body(ent_hbm, sums_hbm, ent_v, sums_v):
    B, L = ent_hbm.shape
    rpw = B // _NW
    wid = lax.axis_index("s") * _NC + lax.axis_index("c")
    base = wid * rpw

    pltpu.sync_copy(ent_hbm.at[pl.ds(base, rpw)], ent_v)
    lane = lax.iota(jnp.int32, 16)

    def _group(g, carry):
        r0 = g * 16
        acc = jnp.zeros((16,), jnp.float32)
        for j in range(16):
            v = ent_v[r0 + j, pl.ds(0, 16)] + ent_v[r0 + j, pl.ds(16, 16)]
            acc = jnp.where(lane == j, jnp.sum(v), acc)
        sums_v[pl.ds(r0, 16)] = acc
        return carry

    lax.fori_loop(0, rpw // 16, _group, 0)
    pltpu.sync_copy(sums_v, sums_hbm.at[pl.ds(base, rpw)])


def kernel(messages, apply_noise, entropy):
    B, L, V = messages.shape
    rpw = B // _NW

    sc_sums = pl.kernel(
        _sc_sums_body,
        out_type=jax.ShapeDtypeStruct((B,), entropy.dtype),
        mesh=plsc.VectorSubcoreMesh(core_axis_name="c", subcore_axis_name="s"),
        compiler_params=pltpu.CompilerParams(needs_layout_passes=False),
        scratch_types=[
            pltpu.VMEM((rpw, L), jnp.float32),
            pltpu.VMEM((rpw,), jnp.float32),
        ],
    )
    sums = sc_sums(entropy)

    out, sent, snn, ment, mnn = pl.pallas_call(
        _tc_copy_body,
        grid=(B // _TB,),
        in_specs=[
            pl.BlockSpec((_TB, L, V), lambda i: (i, 0, 0)),
            pl.BlockSpec((_TB, L), lambda i: (i, 0)),
            pl.BlockSpec((_TB,), lambda i: (i,)),
        ],
        out_specs=[
            pl.BlockSpec((_TB, L, V), lambda i: (i, 0, 0)),
            pl.BlockSpec((_TB, L), lambda i: (i, 0)),
            pl.BlockSpec((_TB, L), lambda i: (i, 0)),
            pl.BlockSpec((_TB,), lambda i: (i,)),
            pl.BlockSpec((_TB,), lambda i: (i,)),
        ],
        out_shape=[
            jax.ShapeDtypeStruct((B, L, V), messages.dtype),
            jax.ShapeDtypeStruct((B, L), entropy.dtype),
            jax.ShapeDtypeStruct((B, L), entropy.dtype),
            jax.ShapeDtypeStruct((B,), entropy.dtype),
            jax.ShapeDtypeStruct((B,), entropy.dtype),
        ],
    )(messages, entropy, sums)

    return (out, ment, sent, mnn, snn)


# speedup vs baseline: 1.1926x; 1.1926x over previous
import jax
import jax.numpy as jnp
from jax.experimental import pallas as pl

_TB = 256


def _body(msg_ref, ent_ref, out_ref, ment_ref, sent_ref, mnn_ref, snn_ref):
    out_ref[...] = msg_ref[...]
    e = ent_ref[...]
    s = jnp.sum(e, axis=1)
    ment_ref[...] = s
    sent_ref[...] = e
    mnn_ref[...] = s
    snn_ref[...] = e


def kernel(messages, apply_noise, entropy):
    B, L, V = messages.shape
    out, ment, sent, mnn, snn = pl.pallas_call(
        _body,
        grid=(B // _TB,),
        in_specs=[
            pl.BlockSpec((_TB, L, V), lambda i: (i, 0, 0)),
            pl.BlockSpec((_TB, L), lambda i: (i, 0)),
        ],
        out_specs=[
            pl.BlockSpec((_TB, L, V), lambda i: (i, 0, 0)),
            pl.BlockSpec((_TB,), lambda i: (i,)),
            pl.BlockSpec((_TB, L), lambda i: (i, 0)),
            pl.BlockSpec((_TB,), lambda i: (i,)),
            pl.BlockSpec((_TB, L), lambda i: (i, 0)),
        ],
        out_shape=[
            jax.ShapeDtypeStruct((B, L, V), messages.dtype),
            jax.ShapeDtypeStruct((B,), entropy.dtype),
            jax.ShapeDtypeStruct((B, L), entropy.dtype),
            jax.ShapeDtypeStruct((B,), entropy.dtype),
            jax.ShapeDtypeStruct((B, L), entropy.dtype),
        ],
    )(messages, entropy)
    return (out, ment, sent, mnn, snn)


# speedup vs baseline: 1.2471x; 1.0457x over previous
import jax
import jax.numpy as jnp
from jax.experimental import pallas as pl

_TB = 256


def _body(msg_ref, ent_ref, out_ref, ment_ref, mnn_ref):
    out_ref[...] = msg_ref[...]
    s = jnp.sum(ent_ref[...], axis=1)
    ment_ref[...] = s
    mnn_ref[...] = s


def kernel(messages, apply_noise, entropy):
    B, L, V = messages.shape
    out, ment, mnn = pl.pallas_call(
        _body,
        grid=(B // _TB,),
        in_specs=[
            pl.BlockSpec((_TB, L, V), lambda i: (i, 0, 0)),
            pl.BlockSpec((_TB, L), lambda i: (i, 0)),
        ],
        out_specs=[
            pl.BlockSpec((_TB, L, V), lambda i: (i, 0, 0)),
            pl.BlockSpec((_TB,), lambda i: (i,)),
            pl.BlockSpec((_TB,), lambda i: (i,)),
        ],
        out_shape=[
            jax.ShapeDtypeStruct((B, L, V), messages.dtype),
            jax.ShapeDtypeStruct((B,), entropy.dtype),
            jax.ShapeDtypeStruct((B,), entropy.dtype),
        ],
    )(messages, entropy)
    sent = entropy + jnp.zeros((), entropy.dtype)
    snn = entropy + jnp.zeros((), entropy.dtype)
    return (out, ment, sent, mnn, snn)
